# baseline (speedup 1.0000x reference)
import functools

import jax
import jax.numpy as jnp
from jax import lax
from jax.experimental import pallas as pl
from jax.experimental.pallas import tpu as pltpu
from jax.experimental.pallas import tpu_sc as plsc

N_NODES = 10000
N_EDGES = 160000
D = 256
H = D // 2
NS = 16
EPT = N_EDGES // NS
B = 128
CH = 8
NCH = 10
KCH = NCH + 2
NPAD = 10240
RPT = NPAD // NS


def _sc_segment_sum(x2, idx_all, zer):
    mesh = plsc.VectorSubcoreMesh(core_axis_name="c", subcore_axis_name="s")

    @functools.partial(
        pl.kernel,
        out_type=jax.ShapeDtypeStruct((2, NPAD, H), jnp.float32),
        mesh=mesh,
        scratch_types=[
            pltpu.VMEM((CH, 2, B), jnp.int32),
            pltpu.VMEM((CH, 2, B), jnp.int32),
            pltpu.VMEM((B, H), jnp.float32),
            pltpu.VMEM((B, H), jnp.float32),
            pltpu.VMEM_SHARED((NPAD, H), jnp.float32),
            pltpu.SemaphoreType.DMA,
            pltpu.SemaphoreType.DMA,
            pltpu.SemaphoreType.DMA,
            pltpu.SemaphoreType.DMA,
        ],
    )
    def sc_accum(x2_hbm, idx_hbm, zer_hbm, out_hbm,
                 ib0, ib1, buf0, buf1, s_sh, si0, si1, sg0, sg1):
        c = lax.axis_index("c")
        t = lax.axis_index("s")
        pltpu.sync_copy(zer_hbm, s_sh.at[pl.ds(t * RPT, RPT)])
        plsc.subcore_barrier()

        pltpu.sync_copy(idx_hbm.at[c, t, 0], ib0)
        pltpu.async_copy(x2_hbm.at[ib0.at[0, 0]], buf0, sg0)
        pltpu.async_copy(idx_hbm.at[c, t, 1], ib1, si1)

        def chunk_step(cc, ib_a, si_a, ib_b, si_b):
            for j in range(CH):
                buf_a, sg_a = (buf0, sg0) if j % 2 == 0 else (buf1, sg1)
                buf_b, sg_b = (buf1, sg1) if j % 2 == 0 else (buf0, sg0)
                if j < CH - 1:
                    gidx_next = ib_a.at[j + 1, 0]
                else:
                    pltpu.make_async_copy(
                        idx_hbm.at[c, t, cc + 1], ib_b, si_b).wait()
                    gidx_next = ib_b.at[0, 0]
                pltpu.async_copy(x2_hbm.at[gidx_next], buf_b, sg_b)
                pltpu.make_async_copy(
                    x2_hbm.at[ib_a.at[j, 0]], buf_a, sg_a).wait()
                pltpu.sync_copy(buf_a, s_sh.at[ib_a.at[j, 1]], add=True)
            pltpu.async_copy(idx_hbm.at[c, t, cc + 2], ib_a, si_a)

        def step(m, carry):
            c0 = 2 * m
            chunk_step(c0, ib0, si0, ib1, si1)
            chunk_step(c0 + 1, ib1, si1, ib0, si0)
            return carry

        lax.fori_loop(0, NCH // 2, step, 0)
        pltpu.make_async_copy(x2_hbm.at[ib0.at[0, 0]], buf0, sg0).wait()
        pltpu.make_async_copy(idx_hbm.at[c, t, NCH + 1], ib1, si1).wait()
        plsc.subcore_barrier()
        pltpu.sync_copy(s_sh.at[pl.ds(t * RPT, RPT)],
                        out_hbm.at[c, pl.ds(t * RPT, RPT)])

    return sc_accum(x2, idx_all, zer)


def _tc_finish(x, s0, s1, wt, bias2):
    blk = 2000
    grid = (N_NODES // blk,)

    def body(x_ref, s0_ref, s1_ref, wt_ref, b_ref, o_ref):
        xs = x_ref[...] * jnp.concatenate([s0_ref[...], s1_ref[...]], axis=-1)
        o_ref[...] = (jnp.dot(xs, wt_ref[...],
                              preferred_element_type=jnp.float32)
                      + b_ref[...])

    return pl.pallas_call(
        body,
        grid=grid,
        in_specs=[
            pl.BlockSpec((blk, D), lambda i: (i, 0)),
            pl.BlockSpec((blk, H), lambda i: (i, 0)),
            pl.BlockSpec((blk, H), lambda i: (i, 0)),
            pl.BlockSpec((D, D), lambda i: (0, 0)),
            pl.BlockSpec((1, D), lambda i: (0, 0)),
        ],
        out_specs=pl.BlockSpec((blk, D), lambda i: (i, 0)),
        out_shape=jax.ShapeDtypeStruct((N_NODES, D), jnp.float32),
    )(x, s0, s1, wt, bias2)


def kernel(x, edge_index, weight, bias):
    row = edge_index[0].astype(jnp.int32)
    col = edge_index[1].astype(jnp.int32)
    x2 = x.reshape(2 * N_NODES, H)
    npad = KCH * CH * B - EPT
    colp = jnp.concatenate(
        [col.reshape(NS, EPT),
         jnp.zeros((NS, npad), jnp.int32)], axis=1)
    rowp = jnp.concatenate(
        [row.reshape(NS, EPT),
         jnp.full((NS, npad), NPAD - 1, jnp.int32)], axis=1)
    gidx = jnp.stack([colp * 2, colp * 2 + 1])
    sidx = jnp.broadcast_to(rowp, (2, NS, KCH * CH * B))
    idx_all = jnp.stack(
        [gidx.reshape(2, NS, KCH, CH, B), sidx.reshape(2, NS, KCH, CH, B)],
        axis=4)
    zer = jnp.zeros((RPT, H), dtype=jnp.float32)

    s = _sc_segment_sum(x2, idx_all, zer)

    wt = weight.T
    bias2 = bias[None, :]
    return _tc_finish(x, s[0], s[1], wt, bias2)


# speedup vs baseline: 1.6033x; 1.6033x over previous
import functools

import jax
import jax.numpy as jnp
from jax import lax
from jax.experimental import pallas as pl
from jax.experimental.pallas import tpu as pltpu
from jax.experimental.pallas import tpu_sc as plsc

N_NODES = 10000
N_EDGES = 160000
D = 256
H = D // 2
NS = 16
NL = 16
EPT = N_EDGES // NS
B = 128
KR = 79
KB = KR + 1
NPAD = 10112
RPT = NPAD // NS


def _sc_segment_sum(x2, idx_packed, zer):
    mesh = plsc.VectorSubcoreMesh(core_axis_name="c", subcore_axis_name="s")

    @functools.partial(
        pl.kernel,
        out_type=jax.ShapeDtypeStruct((2, NPAD, H), jnp.float32),
        mesh=mesh,
        scratch_types=[
            pltpu.VMEM((B,), jnp.int32),
            pltpu.VMEM((B,), jnp.int32),
            pltpu.VMEM((B,), jnp.int32),
            pltpu.VMEM((B,), jnp.int32),
            pltpu.VMEM((B,), jnp.int32),
            pltpu.VMEM((B,), jnp.int32),
            pltpu.VMEM((B, H), jnp.float32),
            pltpu.VMEM((B, H), jnp.float32),
            pltpu.VMEM_SHARED((NPAD, H), jnp.float32),
            pltpu.SemaphoreType.DMA,
            pltpu.SemaphoreType.DMA,
            pltpu.SemaphoreType.DMA,
            pltpu.SemaphoreType.DMA,
        ],
    )
    def sc_accum(x2_hbm, idx_hbm, zer_hbm, out_hbm,
                 ib0, ib1, ga0, ga1, ra0, ra1, buf0, buf1, s_sh,
                 si0, si1, sg0, sg1):
        c = lax.axis_index("c")
        t = lax.axis_index("s")
        pltpu.sync_copy(zer_hbm, s_sh.at[pl.ds(t * RPT, RPT)])
        plsc.subcore_barrier()

        def unpack(ib, ga, ra):
            for v in range(B // NL):
                w = ib[pl.ds(NL * v, NL)]
                ga[pl.ds(NL * v, NL)] = (w & 0x1FFFF) + c
                ra[pl.ds(NL * v, NL)] = lax.shift_right_logical(w, 17)

        pltpu.sync_copy(idx_hbm.at[t, 0], ib0)
        unpack(ib0, ga0, ra0)
        pltpu.async_copy(x2_hbm.at[ga0], buf0, sg0)
        pltpu.async_copy(idx_hbm.at[t, 1], ib1, si1)

        def half_step(k, ib_a, si_a, ga_a, ra_a, buf_a, sg_a,
                      ib_b, si_b, ga_b, ra_b, buf_b, sg_b):
            pltpu.make_async_copy(idx_hbm.at[t, k + 1], ib_b, si_b).wait()
            unpack(ib_b, ga_b, ra_b)
            pltpu.async_copy(x2_hbm.at[ga_b], buf_b, sg_b)
            pltpu.make_async_copy(x2_hbm.at[ga_a], buf_a, sg_a).wait()
            pltpu.sync_copy(buf_a, s_sh.at[ra_a], add=True)
            pltpu.async_copy(idx_hbm.at[t, k + 2], ib_a, si_a)

        def step(j, carry):
            k0 = 2 * j
            half_step(k0, ib0, si0, ga0, ra0, buf0, sg0,
                      ib1, si1, ga1, ra1, buf1, sg1)
            half_step(k0 + 1, ib1, si1, ga1, ra1, buf1, sg1,
                      ib0, si0, ga0, ra0, buf0, sg0)
            return carry

        lax.fori_loop(0, (KR - 1) // 2, step, 0)
        pltpu.make_async_copy(x2_hbm.at[ga0], buf0, sg0).wait()
        pltpu.sync_copy(buf0, s_sh.at[ra0], add=True)
        pltpu.make_async_copy(idx_hbm.at[t, KR], ib1, si1).wait()
        plsc.subcore_barrier()
        pltpu.sync_copy(s_sh.at[pl.ds(t * RPT, RPT)],
                        out_hbm.at[c, pl.ds(t * RPT, RPT)])

    return sc_accum(x2, idx_packed, zer)


def _tc_finish(x, s0, s1, wt, bias2):
    blk = 2000
    grid = (N_NODES // blk,)

    def body(x_ref, s0_ref, s1_ref, wt_ref, b_ref, o_ref):
        xs = x_ref[...] * jnp.concatenate([s0_ref[...], s1_ref[...]], axis=-1)
        o_ref[...] = (jnp.dot(xs, wt_ref[...],
                              preferred_element_type=jnp.float32)
                      + b_ref[...])

    return pl.pallas_call(
        body,
        grid=grid,
        in_specs=[
            pl.BlockSpec((blk, D), lambda i: (i, 0)),
            pl.BlockSpec((blk, H), lambda i: (i, 0)),
            pl.BlockSpec((blk, H), lambda i: (i, 0)),
            pl.BlockSpec((D, D), lambda i: (0, 0)),
            pl.BlockSpec((1, D), lambda i: (0, 0)),
        ],
        out_specs=pl.BlockSpec((blk, D), lambda i: (i, 0)),
        out_shape=jax.ShapeDtypeStruct((N_NODES, D), jnp.float32),
    )(x, s0, s1, wt, bias2)


def kernel(x, edge_index, weight, bias):
    row = edge_index[0].astype(jnp.int32)
    col = edge_index[1].astype(jnp.int32)
    x2 = x.reshape(2 * N_NODES, H)
    packed = (row << 17) | (col << 1)
    pad = jnp.full((NS, KB * B - EPT), (NPAD - 1) << 17, jnp.int32)
    idx_packed = jnp.concatenate(
        [packed.reshape(NS, EPT), pad], axis=1).reshape(NS, KB, B)
    zer = jnp.zeros((RPT, H), dtype=jnp.float32)

    s = _sc_segment_sum(x2, idx_packed, zer)

    wt = weight.T
    bias2 = bias[None, :]
    return _tc_finish(x, s[0], s[1], wt, bias2)
